# baseline (speedup 1.0000x reference)
import functools

import jax
import jax.numpy as jnp
import numpy as np
from jax import lax
from jax.experimental import pallas as pl
from jax.experimental.pallas import tpu as pltpu
from jax.experimental.pallas import tpu_sc as plsc

N_NODES = 10000
N_EDGES = 320000
D = 128

NC = 2
NS = 16

EB = 80
E_PER_TILE_AGG = N_EDGES // NC // NS
CHUNKS_AGG = E_PER_TILE_AGG // EB

E_PER_TILE_DEG = N_EDGES // NS
CHUNKS_DEG = E_PER_TILE_DEG // EB

N_PAD = 10112
ROWS_PER_TILE = N_PAD // NS

_mesh = plsc.VectorSubcoreMesh(
    core_axis_name="c", subcore_axis_name="s", num_cores=NC, num_subcores=NS
)


def _deg_body(ei_ref, ones_ref, zeros_ref, out_ref, ibuf, val, acc, sem):
    cid = lax.axis_index("c")
    sid = lax.axis_index("s")
    row0 = sid * ROWS_PER_TILE

    pltpu.sync_copy(zeros_ref, acc.at[pl.ds(row0, ROWS_PER_TILE)])
    pltpu.sync_copy(ones_ref, val)
    pltpu.sync_copy(ei_ref.at[cid, sid], ibuf)
    plsc.subcore_barrier()

    DEPTH = 8

    def body(j, carry):
        pltpu.async_copy(val, acc.at[ibuf.at[j]], sem, add=True)
        pl.when(j >= DEPTH)(
            lambda: pltpu.make_async_copy(val, acc.at[ibuf.at[j]], sem).wait())
        return carry

    lax.fori_loop(0, CHUNKS_DEG, body, 0)
    for j in range(DEPTH):
        pltpu.make_async_copy(val, acc.at[ibuf.at[j]], sem).wait()
    plsc.subcore_barrier()
    pltpu.sync_copy(
        acc.at[pl.ds(row0, ROWS_PER_TILE)],
        out_ref.at[cid, pl.ds(row0, ROWS_PER_TILE)],
    )


_deg_kernel = pl.kernel(
    _deg_body,
    out_type=jax.ShapeDtypeStruct((NC, N_PAD, 8), jnp.float32),
    mesh=_mesh,
    scratch_types=[
        pltpu.VMEM((CHUNKS_DEG, EB), jnp.int32),
        pltpu.VMEM((EB, 8), jnp.float32),
        pltpu.VMEM_SHARED((N_PAD, 8), jnp.float32),
        pltpu.SemaphoreType.DMA,
    ],
)


def _agg_body(g_ref, src_ref, dst_ref, zeros_ref, out_ref, sbuf, dbuf,
              vbuf0, vbuf1, acc, gsem0, gsem1, ssem0, ssem1):
    cid = lax.axis_index("c")
    sid = lax.axis_index("s")
    row0 = sid * ROWS_PER_TILE

    pltpu.sync_copy(zeros_ref, acc.at[pl.ds(row0, ROWS_PER_TILE)])
    pltpu.sync_copy(src_ref.at[cid, sid], sbuf)
    pltpu.sync_copy(dst_ref.at[cid, sid], dbuf)
    plsc.subcore_barrier()

    vbufs = (vbuf0, vbuf1)
    gsems = (gsem0, gsem1)
    ssems = (ssem0, ssem1)

    def start_g(j, b):
        pltpu.async_copy(g_ref.at[sbuf.at[pl.ds(j * EB, EB)]],
                         vbufs[b], gsems[b])

    def wait_g(j, b):
        pltpu.make_async_copy(g_ref.at[sbuf.at[pl.ds(j * EB, EB)]],
                              vbufs[b], gsems[b]).wait()

    def start_s(j, b):
        pltpu.async_copy(vbufs[b], acc.at[dbuf.at[j]], ssems[b], add=True)

    def wait_s(j, b):
        pltpu.make_async_copy(vbufs[b], acc.at[dbuf.at[j]], ssems[b]).wait()

    start_g(0, 0)

    def body(p, carry):
        j0 = 2 * p
        j1 = j0 + 1
        wait_g(j0, 0)
        start_s(j0, 0)
        pl.when(p > 0)(lambda: wait_s(j0 - 1, 1))
        pl.when(j1 < CHUNKS_AGG)(lambda: start_g(j1, 1))

        @pl.when(j1 < CHUNKS_AGG)
        def _odd():
            wait_g(j1, 1)
            start_s(j1, 1)
            wait_s(j0, 0)
            pl.when(j1 + 1 < CHUNKS_AGG)(lambda: start_g(j1 + 1, 0))

        return carry

    lax.fori_loop(0, (CHUNKS_AGG + 1) // 2, body, 0)
    wait_s(CHUNKS_AGG - 1, (CHUNKS_AGG - 1) % 2)
    plsc.subcore_barrier()
    pltpu.sync_copy(
        acc.at[pl.ds(row0, ROWS_PER_TILE)],
        out_ref.at[cid, pl.ds(row0, ROWS_PER_TILE)],
    )


_agg_kernel = pl.kernel(
    _agg_body,
    out_type=jax.ShapeDtypeStruct((NC, N_PAD, D), jnp.float32),
    mesh=_mesh,
    scratch_types=[
        pltpu.VMEM((E_PER_TILE_AGG,), jnp.int32),
        pltpu.VMEM((CHUNKS_AGG, EB), jnp.int32),
        pltpu.VMEM((EB, D), jnp.float32),
        pltpu.VMEM((EB, D), jnp.float32),
        pltpu.VMEM_SHARED((N_PAD, D), jnp.float32),
        pltpu.SemaphoreType.DMA,
        pltpu.SemaphoreType.DMA,
        pltpu.SemaphoreType.DMA,
        pltpu.SemaphoreType.DMA,
    ],
)


RB = 2000
_GRID = (N_NODES // RB,)


def _norms(deg_ref):
    d = deg_ref[...]
    deg_s = jnp.sum(d[0], axis=1)
    deg_d = jnp.sum(d[1], axis=1)
    ns = lax.rsqrt(jnp.maximum(deg_s, 1.0))
    nd = lax.rsqrt(jnp.maximum(deg_d, 1.0))
    return ns, nd


def _mm_body(x_ref, w_ref, deg_ref, o_ref):
    ns, _ = _norms(deg_ref)
    o_ref[...] = ns[:, None] * jnp.dot(x_ref[...], w_ref[...],
                                       preferred_element_type=jnp.float32)


def _mid_body(a_ref, deg_ref, b_ref, w_ref, o_ref):
    ns, nd = _norms(deg_ref)
    a = a_ref[0] + a_ref[1]
    h = jax.nn.relu(nd[:, None] * a + b_ref[...][None, :])
    o_ref[...] = jnp.dot(ns[:, None] * h, w_ref[...],
                         preferred_element_type=jnp.float32)


def _final_body(a_ref, deg_ref, b_ref, o_ref):
    _, nd = _norms(deg_ref)
    a = a_ref[0] + a_ref[1]
    o_ref[...] = nd[:, None] * a + b_ref[...][None, :]


_row_spec = pl.BlockSpec((RB, D), lambda i: (i, 0))
_deg_spec = pl.BlockSpec((NC, RB, 8), lambda i: (0, i, 0))
_acc_spec = pl.BlockSpec((NC, RB, D), lambda i: (0, i, 0))
_w_spec = pl.BlockSpec((D, D), lambda i: (0, 0))
_b_spec = pl.BlockSpec((D,), lambda i: (0,))
_out_f32 = jax.ShapeDtypeStruct((N_NODES, D), jnp.float32)

_tc_matmul = pl.pallas_call(
    _mm_body, grid=_GRID, in_specs=[_row_spec, _w_spec, _deg_spec],
    out_specs=_row_spec, out_shape=_out_f32)
_tc_mid = pl.pallas_call(
    _mid_body, grid=_GRID, in_specs=[_acc_spec, _deg_spec, _b_spec, _w_spec],
    out_specs=_row_spec, out_shape=_out_f32)
_tc_final = pl.pallas_call(
    _final_body, grid=_GRID, in_specs=[_acc_spec, _deg_spec, _b_spec],
    out_specs=_row_spec, out_shape=_out_f32)


_ONES_PAT = np.zeros((EB, 8), np.float32)
_ONES_PAT[:, 0] = 1.0


def kernel(x, edge_index, W1, b1, W2, b2, W3, b3):
    ei = edge_index.astype(jnp.int32)
    ei_deg = ei.reshape(2, NS, CHUNKS_DEG, EB)
    src_agg = ei[0].reshape(NC, NS, E_PER_TILE_AGG)
    dst_agg = ei[1].reshape(NC, NS, CHUNKS_AGG, EB)
    ones_pat = jnp.asarray(_ONES_PAT)
    zeros8 = jnp.zeros((ROWS_PER_TILE, 8), jnp.float32)
    zeros128 = jnp.zeros((ROWS_PER_TILE, D), jnp.float32)

    degs = _deg_kernel(ei_deg, ones_pat, zeros8)
    g1 = _tc_matmul(x, W1, degs)
    a1 = _agg_kernel(g1, src_agg, dst_agg, zeros128)
    g2 = _tc_mid(a1, degs, b1, W2)
    a2 = _agg_kernel(g2, src_agg, dst_agg, zeros128)
    g3 = _tc_mid(a2, degs, b2, W3)
    a3 = _agg_kernel(g3, src_agg, dst_agg, zeros128)
    return _tc_final(a3, degs, b3)


# speedup vs baseline: 1.0013x; 1.0013x over previous
import functools

import jax
import jax.numpy as jnp
import numpy as np
from jax import lax
from jax.experimental import pallas as pl
from jax.experimental.pallas import tpu as pltpu
from jax.experimental.pallas import tpu_sc as plsc

N_NODES = 10000
N_EDGES = 320000
D = 128

NC = 2
NS = 16

EB = 80
E_PER_TILE_AGG = N_EDGES // NC // NS
CHUNKS_AGG = E_PER_TILE_AGG // EB

DEG_EB = 100
E_PER_TILE_DEG = N_EDGES // NS
CHUNKS_DEG = E_PER_TILE_DEG // DEG_EB

N_PAD = 10112
ROWS_PER_TILE = N_PAD // NS

_mesh = plsc.VectorSubcoreMesh(
    core_axis_name="c", subcore_axis_name="s", num_cores=NC, num_subcores=NS
)


def _deg_body(ei_ref, ones_ref, zeros_ref, out_ref, ibuf, val, acc, sem):
    cid = lax.axis_index("c")
    sid = lax.axis_index("s")
    row0 = sid * ROWS_PER_TILE

    pltpu.sync_copy(zeros_ref, acc.at[pl.ds(row0, ROWS_PER_TILE)])
    pltpu.sync_copy(ones_ref, val)
    pltpu.sync_copy(ei_ref.at[cid, sid], ibuf)
    plsc.subcore_barrier()

    DEPTH = 8

    def body(j, carry):
        pltpu.async_copy(val, acc.at[ibuf.at[j]], sem, add=True)
        pl.when(j >= DEPTH)(
            lambda: pltpu.make_async_copy(val, acc.at[ibuf.at[j]], sem).wait())
        return carry

    lax.fori_loop(0, CHUNKS_DEG, body, 0)
    for j in range(DEPTH):
        pltpu.make_async_copy(val, acc.at[ibuf.at[j]], sem).wait()
    plsc.subcore_barrier()
    pltpu.sync_copy(
        acc.at[pl.ds(row0, ROWS_PER_TILE)],
        out_ref.at[cid, pl.ds(row0, ROWS_PER_TILE)],
    )


_deg_kernel = pl.kernel(
    _deg_body,
    out_type=jax.ShapeDtypeStruct((NC, N_PAD, 8), jnp.float32),
    mesh=_mesh,
    scratch_types=[
        pltpu.VMEM((CHUNKS_DEG, DEG_EB), jnp.int32),
        pltpu.VMEM((DEG_EB, 8), jnp.float32),
        pltpu.VMEM_SHARED((N_PAD, 8), jnp.float32),
        pltpu.SemaphoreType.DMA,
    ],
)


def _agg_body(g_ref, src_ref, dst_ref, zeros_ref, out_ref, sbuf, dbuf,
              vbuf0, vbuf1, acc, gsem0, gsem1, ssem0, ssem1):
    cid = lax.axis_index("c")
    sid = lax.axis_index("s")
    row0 = sid * ROWS_PER_TILE

    pltpu.sync_copy(zeros_ref, acc.at[pl.ds(row0, ROWS_PER_TILE)])
    pltpu.sync_copy(src_ref.at[cid, sid], sbuf)
    pltpu.sync_copy(dst_ref.at[cid, sid], dbuf)
    plsc.subcore_barrier()

    vbufs = (vbuf0, vbuf1)
    gsems = (gsem0, gsem1)
    ssems = (ssem0, ssem1)

    def start_g(j, b):
        pltpu.async_copy(g_ref.at[sbuf.at[pl.ds(j * EB, EB)]],
                         vbufs[b], gsems[b])

    def wait_g(j, b):
        pltpu.make_async_copy(g_ref.at[sbuf.at[pl.ds(j * EB, EB)]],
                              vbufs[b], gsems[b]).wait()

    def start_s(j, b):
        pltpu.async_copy(vbufs[b], acc.at[dbuf.at[j]], ssems[b], add=True)

    def wait_s(j, b):
        pltpu.make_async_copy(vbufs[b], acc.at[dbuf.at[j]], ssems[b]).wait()

    start_g(0, 0)

    def body(p, carry):
        j0 = 2 * p
        j1 = j0 + 1
        wait_g(j0, 0)
        start_s(j0, 0)
        pl.when(p > 0)(lambda: wait_s(j0 - 1, 1))
        pl.when(j1 < CHUNKS_AGG)(lambda: start_g(j1, 1))

        @pl.when(j1 < CHUNKS_AGG)
        def _odd():
            wait_g(j1, 1)
            start_s(j1, 1)
            wait_s(j0, 0)
            pl.when(j1 + 1 < CHUNKS_AGG)(lambda: start_g(j1 + 1, 0))

        return carry

    lax.fori_loop(0, (CHUNKS_AGG + 1) // 2, body, 0)
    wait_s(CHUNKS_AGG - 1, (CHUNKS_AGG - 1) % 2)
    plsc.subcore_barrier()
    pltpu.sync_copy(
        acc.at[pl.ds(row0, ROWS_PER_TILE)],
        out_ref.at[cid, pl.ds(row0, ROWS_PER_TILE)],
    )


_agg_kernel = pl.kernel(
    _agg_body,
    out_type=jax.ShapeDtypeStruct((NC, N_PAD, D), jnp.float32),
    mesh=_mesh,
    scratch_types=[
        pltpu.VMEM((E_PER_TILE_AGG,), jnp.int32),
        pltpu.VMEM((CHUNKS_AGG, EB), jnp.int32),
        pltpu.VMEM((EB, D), jnp.float32),
        pltpu.VMEM((EB, D), jnp.float32),
        pltpu.VMEM_SHARED((N_PAD, D), jnp.float32),
        pltpu.SemaphoreType.DMA,
        pltpu.SemaphoreType.DMA,
        pltpu.SemaphoreType.DMA,
        pltpu.SemaphoreType.DMA,
    ],
)


RB = 5000
_GRID = (N_NODES // RB,)


def _norms(deg_ref):
    d = deg_ref[...]
    deg_s = jnp.sum(d[0], axis=1)
    deg_d = jnp.sum(d[1], axis=1)
    ns = lax.rsqrt(jnp.maximum(deg_s, 1.0))
    nd = lax.rsqrt(jnp.maximum(deg_d, 1.0))
    return ns, nd


def _mm_body(x_ref, w_ref, deg_ref, o_ref):
    ns, _ = _norms(deg_ref)
    o_ref[...] = ns[:, None] * jnp.dot(x_ref[...], w_ref[...],
                                       preferred_element_type=jnp.float32)


def _mid_body(a_ref, deg_ref, b_ref, w_ref, o_ref):
    ns, nd = _norms(deg_ref)
    a = a_ref[0] + a_ref[1]
    h = jax.nn.relu(nd[:, None] * a + b_ref[...][None, :])
    o_ref[...] = jnp.dot(ns[:, None] * h, w_ref[...],
                         preferred_element_type=jnp.float32)


def _final_body(a_ref, deg_ref, b_ref, o_ref):
    _, nd = _norms(deg_ref)
    a = a_ref[0] + a_ref[1]
    o_ref[...] = nd[:, None] * a + b_ref[...][None, :]


_row_spec = pl.BlockSpec((RB, D), lambda i: (i, 0))
_deg_spec = pl.BlockSpec((NC, RB, 8), lambda i: (0, i, 0))
_acc_spec = pl.BlockSpec((NC, RB, D), lambda i: (0, i, 0))
_w_spec = pl.BlockSpec((D, D), lambda i: (0, 0))
_b_spec = pl.BlockSpec((D,), lambda i: (0,))
_out_f32 = jax.ShapeDtypeStruct((N_NODES, D), jnp.float32)

_tc_matmul = pl.pallas_call(
    _mm_body, grid=_GRID, in_specs=[_row_spec, _w_spec, _deg_spec],
    out_specs=_row_spec, out_shape=_out_f32)
_tc_mid = pl.pallas_call(
    _mid_body, grid=_GRID, in_specs=[_acc_spec, _deg_spec, _b_spec, _w_spec],
    out_specs=_row_spec, out_shape=_out_f32)
_tc_final = pl.pallas_call(
    _final_body, grid=_GRID, in_specs=[_acc_spec, _deg_spec, _b_spec],
    out_specs=_row_spec, out_shape=_out_f32)


_ONES_PAT = np.zeros((DEG_EB, 8), np.float32)
_ONES_PAT[:, 0] = 1.0


def kernel(x, edge_index, W1, b1, W2, b2, W3, b3):
    ei = edge_index.astype(jnp.int32)
    ei_deg = ei.reshape(2, NS, CHUNKS_DEG, DEG_EB)
    src_agg = ei[0].reshape(NC, NS, E_PER_TILE_AGG)
    dst_agg = ei[1].reshape(NC, NS, CHUNKS_AGG, EB)
    ones_pat = jnp.asarray(_ONES_PAT)
    zeros8 = jnp.zeros((ROWS_PER_TILE, 8), jnp.float32)
    zeros128 = jnp.zeros((ROWS_PER_TILE, D), jnp.float32)

    degs = _deg_kernel(ei_deg, ones_pat, zeros8)
    g1 = _tc_matmul(x, W1, degs)
    a1 = _agg_kernel(g1, src_agg, dst_agg, zeros128)
    g2 = _tc_mid(a1, degs, b1, W2)
    a2 = _agg_kernel(g2, src_agg, dst_agg, zeros128)
    g3 = _tc_mid(a2, degs, b2, W3)
    a3 = _agg_kernel(g3, src_agg, dst_agg, zeros128)
    return _tc_final(a3, degs, b3)


# speedup vs baseline: 1.0197x; 1.0183x over previous
import functools

import jax
import jax.numpy as jnp
import numpy as np
from jax import lax
from jax.experimental import pallas as pl
from jax.experimental.pallas import tpu as pltpu
from jax.experimental.pallas import tpu_sc as plsc

N_NODES = 10000
N_EDGES = 320000
D = 128

NC = 2
NS = 16

EB = 80
E_PER_TILE_AGG = N_EDGES // NC // NS
CHUNKS_AGG = E_PER_TILE_AGG // EB

DEG_EB = 100
E_PER_TILE_DEG = N_EDGES // NS
CHUNKS_DEG = E_PER_TILE_DEG // DEG_EB

N_PAD = 10112
ROWS_PER_TILE = N_PAD // NS

_mesh = plsc.VectorSubcoreMesh(
    core_axis_name="c", subcore_axis_name="s", num_cores=NC, num_subcores=NS
)


def _deg_body(ei_ref, ones_ref, zeros_ref, out_ref, ibuf, val, acc, sem):
    cid = lax.axis_index("c")
    sid = lax.axis_index("s")
    row0 = sid * ROWS_PER_TILE

    zcp = pltpu.async_copy(zeros_ref, acc.at[pl.ds(row0, ROWS_PER_TILE)], sem)
    pltpu.sync_copy(ones_ref, val)
    pltpu.sync_copy(ei_ref.at[cid, sid], ibuf)
    zcp.wait()
    plsc.subcore_barrier()

    DEPTH = 8

    def body(j, carry):
        pltpu.async_copy(val, acc.at[ibuf.at[j]], sem, add=True)
        pl.when(j >= DEPTH)(
            lambda: pltpu.make_async_copy(val, acc.at[ibuf.at[j]], sem).wait())
        return carry

    lax.fori_loop(0, CHUNKS_DEG, body, 0)
    for j in range(DEPTH):
        pltpu.make_async_copy(val, acc.at[ibuf.at[j]], sem).wait()
    plsc.subcore_barrier()
    pltpu.sync_copy(
        acc.at[pl.ds(row0, ROWS_PER_TILE)],
        out_ref.at[cid, pl.ds(row0, ROWS_PER_TILE)],
    )


_deg_kernel = pl.kernel(
    _deg_body,
    out_type=jax.ShapeDtypeStruct((NC, N_PAD, 8), jnp.float32),
    mesh=_mesh,
    scratch_types=[
        pltpu.VMEM((CHUNKS_DEG, DEG_EB), jnp.int32),
        pltpu.VMEM((DEG_EB, 8), jnp.float32),
        pltpu.VMEM_SHARED((N_PAD, 8), jnp.float32),
        pltpu.SemaphoreType.DMA,
    ],
)


def _agg_body(g_ref, src_ref, dst_ref, zeros_ref, out_ref, sbuf, dbuf,
              vbuf0, vbuf1, acc, gsem0, gsem1, ssem0, ssem1):
    cid = lax.axis_index("c")
    sid = lax.axis_index("s")
    row0 = sid * ROWS_PER_TILE

    zcp = pltpu.async_copy(zeros_ref, acc.at[pl.ds(row0, ROWS_PER_TILE)], gsem1)
    pltpu.sync_copy(src_ref.at[cid, sid], sbuf)
    pltpu.sync_copy(dst_ref.at[cid, sid], dbuf)
    zcp.wait()

    vbufs = (vbuf0, vbuf1)
    gsems = (gsem0, gsem1)
    ssems = (ssem0, ssem1)

    def start_g(j, b):
        pltpu.async_copy(g_ref.at[sbuf.at[pl.ds(j * EB, EB)]],
                         vbufs[b], gsems[b])

    def wait_g(j, b):
        pltpu.make_async_copy(g_ref.at[sbuf.at[pl.ds(j * EB, EB)]],
                              vbufs[b], gsems[b]).wait()

    def start_s(j, b):
        pltpu.async_copy(vbufs[b], acc.at[dbuf.at[j]], ssems[b], add=True)

    def wait_s(j, b):
        pltpu.make_async_copy(vbufs[b], acc.at[dbuf.at[j]], ssems[b]).wait()

    start_g(0, 0)
    plsc.subcore_barrier()

    def body(p, carry):
        j0 = 2 * p
        j1 = j0 + 1
        wait_g(j0, 0)
        start_s(j0, 0)
        pl.when(p > 0)(lambda: wait_s(j0 - 1, 1))
        pl.when(j1 < CHUNKS_AGG)(lambda: start_g(j1, 1))

        @pl.when(j1 < CHUNKS_AGG)
        def _odd():
            wait_g(j1, 1)
            start_s(j1, 1)
            wait_s(j0, 0)
            pl.when(j1 + 1 < CHUNKS_AGG)(lambda: start_g(j1 + 1, 0))

        return carry

    lax.fori_loop(0, (CHUNKS_AGG + 1) // 2, body, 0)
    wait_s(CHUNKS_AGG - 1, (CHUNKS_AGG - 1) % 2)
    plsc.subcore_barrier()
    pltpu.sync_copy(
        acc.at[pl.ds(row0, ROWS_PER_TILE)],
        out_ref.at[cid, pl.ds(row0, ROWS_PER_TILE)],
    )


_agg_kernel = pl.kernel(
    _agg_body,
    out_type=jax.ShapeDtypeStruct((NC, N_PAD, D), jnp.float32),
    mesh=_mesh,
    scratch_types=[
        pltpu.VMEM((E_PER_TILE_AGG,), jnp.int32),
        pltpu.VMEM((CHUNKS_AGG, EB), jnp.int32),
        pltpu.VMEM((EB, D), jnp.float32),
        pltpu.VMEM((EB, D), jnp.float32),
        pltpu.VMEM_SHARED((N_PAD, D), jnp.float32),
        pltpu.SemaphoreType.DMA,
        pltpu.SemaphoreType.DMA,
        pltpu.SemaphoreType.DMA,
        pltpu.SemaphoreType.DMA,
    ],
)


RB = 5000
_GRID = (N_NODES // RB,)


def _norms(deg_ref):
    d = deg_ref[...]
    deg_s = jnp.sum(d[0], axis=1)
    deg_d = jnp.sum(d[1], axis=1)
    ns = lax.rsqrt(jnp.maximum(deg_s, 1.0))
    nd = lax.rsqrt(jnp.maximum(deg_d, 1.0))
    return ns, nd


def _mm_body(x_ref, w_ref, deg_ref, o_ref):
    ns, _ = _norms(deg_ref)
    o_ref[...] = ns[:, None] * jnp.dot(x_ref[...], w_ref[...],
                                       preferred_element_type=jnp.float32)


def _mid_body(a_ref, deg_ref, b_ref, w_ref, o_ref):
    ns, nd = _norms(deg_ref)
    a = a_ref[0] + a_ref[1]
    h = jax.nn.relu(nd[:, None] * a + b_ref[...][None, :])
    o_ref[...] = jnp.dot(ns[:, None] * h, w_ref[...],
                         preferred_element_type=jnp.float32)


def _final_body(a_ref, deg_ref, b_ref, o_ref):
    _, nd = _norms(deg_ref)
    a = a_ref[0] + a_ref[1]
    o_ref[...] = nd[:, None] * a + b_ref[...][None, :]


_row_spec = pl.BlockSpec((RB, D), lambda i: (i, 0))
_deg_spec = pl.BlockSpec((NC, RB, 8), lambda i: (0, i, 0))
_acc_spec = pl.BlockSpec((NC, RB, D), lambda i: (0, i, 0))
_w_spec = pl.BlockSpec((D, D), lambda i: (0, 0))
_b_spec = pl.BlockSpec((D,), lambda i: (0,))
_out_f32 = jax.ShapeDtypeStruct((N_NODES, D), jnp.float32)

_tc_matmul = pl.pallas_call(
    _mm_body, grid=_GRID, in_specs=[_row_spec, _w_spec, _deg_spec],
    out_specs=_row_spec, out_shape=_out_f32)
_tc_mid = pl.pallas_call(
    _mid_body, grid=_GRID, in_specs=[_acc_spec, _deg_spec, _b_spec, _w_spec],
    out_specs=_row_spec, out_shape=_out_f32)
_tc_final = pl.pallas_call(
    _final_body, grid=_GRID, in_specs=[_acc_spec, _deg_spec, _b_spec],
    out_specs=_row_spec, out_shape=_out_f32)


_ONES_PAT = np.zeros((DEG_EB, 8), np.float32)
_ONES_PAT[:, 0] = 1.0


def kernel(x, edge_index, W1, b1, W2, b2, W3, b3):
    ei = edge_index.astype(jnp.int32)
    ei_deg = ei.reshape(2, NS, CHUNKS_DEG, DEG_EB)
    src_agg = ei[0].reshape(NC, NS, E_PER_TILE_AGG)
    dst_agg = ei[1].reshape(NC, NS, CHUNKS_AGG, EB)
    ones_pat = jnp.asarray(_ONES_PAT)
    zeros8 = jnp.zeros((ROWS_PER_TILE, 8), jnp.float32)
    zeros128 = jnp.zeros((ROWS_PER_TILE, D), jnp.float32)

    degs = _deg_kernel(ei_deg, ones_pat, zeros8)
    g1 = _tc_matmul(x, W1, degs)
    a1 = _agg_kernel(g1, src_agg, dst_agg, zeros128)
    g2 = _tc_mid(a1, degs, b1, W2)
    a2 = _agg_kernel(g2, src_agg, dst_agg, zeros128)
    g3 = _tc_mid(a2, degs, b2, W3)
    a3 = _agg_kernel(g3, src_agg, dst_agg, zeros128)
    return _tc_final(a3, degs, b3)
